# baseline (speedup 1.0000x reference)
import functools

import jax
import jax.numpy as jnp
from jax import lax
from jax.experimental import pallas as pl
from jax.experimental.pallas import tpu as pltpu
from jax.experimental.pallas import tpu_sc as plsc

NUM_HEAD = 100
EMBED_DIM = 256
BATCH = 16384

_info = plsc.get_sparse_core_info()
_NC, _NS, _NL = _info.num_cores, _info.num_subcores, _info.num_lanes
_NW = _NC * _NS
_N0 = 400
_N1 = (BATCH // _NS) - _N0

_mesh = plsc.VectorSubcoreMesh(core_axis_name="c", subcore_axis_name="s")


@functools.partial(
    pl.kernel,
    mesh=_mesh,
    out_type=jax.ShapeDtypeStruct((BATCH, 1, EMBED_DIM), jnp.float32),
    scratch_types=[
        pltpu.VMEM((NUM_HEAD, EMBED_DIM), jnp.float32),
        pltpu.VMEM((max(_N0, _N1),), jnp.int32),
        pltpu.VMEM((1, EMBED_DIM), jnp.float32),
        pltpu.SemaphoreType.DMA,
    ],
)
def _gather_kernel(table_hbm, idx_hbm, out_hbm, table_v, idx_v, dummy_v, sem):
    cidx = lax.axis_index("c")
    sidx = lax.axis_index("s")
    base = jnp.where(cidx == 0, sidx * _N0, _NS * _N0 + sidx * _N1)
    count = jnp.where(cidx == 0, _N0, _N1)

    @pl.when(cidx == 0)
    def _():
        pltpu.sync_copy(idx_hbm.at[pl.ds(sidx * _N0, _N0)], idx_v.at[pl.ds(0, _N0)])

    @pl.when(cidx != 0)
    def _():
        pltpu.sync_copy(
            idx_hbm.at[pl.ds(_NS * _N0 + sidx * _N1, _N1)], idx_v.at[pl.ds(0, _N1)]
        )

    pltpu.sync_copy(table_hbm, table_v)

    def body(g, _):
        lblv = idx_v[pl.ds(g * _NL, _NL)]
        for k in range(_NL):
            pltpu.async_copy(
                table_v.at[lblv[k]],
                out_hbm.at[base + g * _NL + k, 0],
                sem,
            )
        return 0

    lax.fori_loop(0, count // _NL, body, 0)

    def drain(r, _):
        pltpu.make_async_copy(out_hbm.at[pl.ds(base, 1), 0], dummy_v, sem).wait()
        return 0

    lax.fori_loop(0, count, drain, 0)


def kernel(params, labels, head_table):
    del params
    return _gather_kernel(head_table, labels)


# speedup vs baseline: 1.0968x; 1.0968x over previous
import functools

import jax
import jax.numpy as jnp
from jax import lax
from jax.experimental import pallas as pl
from jax.experimental.pallas import tpu as pltpu
from jax.experimental.pallas import tpu_sc as plsc

NUM_HEAD = 100
EMBED_DIM = 256
BATCH = 16384

_info = plsc.get_sparse_core_info()
_NC, _NS = _info.num_cores, _info.num_subcores
_NW = _NC * _NS
_B_PER_W = BATCH // _NW
_CHUNK = 128

_mesh = plsc.VectorSubcoreMesh(core_axis_name="c", subcore_axis_name="s")


@functools.partial(
    pl.kernel,
    mesh=_mesh,
    out_type=jax.ShapeDtypeStruct((BATCH, 1, EMBED_DIM), jnp.float32),
    scratch_types=[
        pltpu.VMEM((NUM_HEAD, EMBED_DIM), jnp.float32),
        pltpu.VMEM((_B_PER_W,), jnp.int32),
        pltpu.VMEM((_CHUNK, EMBED_DIM), jnp.float32),
        pltpu.SemaphoreType.DMA,
    ],
)
def _gather_kernel(table_hbm, idx_hbm, out_hbm, table_v, idx_v, dummy_v, sem):
    cidx = lax.axis_index("c")
    wid = lax.axis_index("s") * _NC + cidx
    base = wid * _B_PER_W

    pltpu.sync_copy(idx_hbm.at[pl.ds(base, _B_PER_W)], idx_v)
    pltpu.sync_copy(table_hbm.at[cidx], table_v)

    _NL = 16

    def body(g, _):
        lblv = idx_v[pl.ds(g * _NL, _NL)]
        for k in range(_NL):
            pltpu.async_copy(
                table_v.at[lblv[k]],
                out_hbm.at[base + g * _NL + k, 0],
                sem,
            )
        return 0

    lax.fori_loop(0, _B_PER_W // _NL, body, 0)
    for i in range(_B_PER_W // _CHUNK):
        pltpu.make_async_copy(
            out_hbm.at[pl.ds(base + i * _CHUNK, _CHUNK), 0], dummy_v, sem
        ).wait()


def kernel(params, labels, head_table):
    del params
    table2 = jnp.broadcast_to(head_table[None], (_NC, NUM_HEAD, EMBED_DIM))
    return _gather_kernel(table2, labels)


# speedup vs baseline: 1.2447x; 1.1348x over previous
import functools

import jax
import jax.numpy as jnp
from jax import lax
from jax.experimental import pallas as pl
from jax.experimental.pallas import tpu as pltpu
from jax.experimental.pallas import tpu_sc as plsc

NUM_HEAD = 100
EMBED_DIM = 256
BATCH = 16384

_info = plsc.get_sparse_core_info()
_NC, _NS = _info.num_cores, _info.num_subcores
_NW = _NC * _NS
_B_PER_W = BATCH // _NW
_CHUNK = 128

_mesh = plsc.VectorSubcoreMesh(core_axis_name="c", subcore_axis_name="s")


@functools.partial(
    pl.kernel,
    mesh=_mesh,
    out_type=jax.ShapeDtypeStruct((BATCH, 1, EMBED_DIM), jnp.float32),
    scratch_types=[
        pltpu.VMEM((NUM_HEAD, EMBED_DIM), jnp.float32),
        pltpu.VMEM((_B_PER_W,), jnp.int32),
        pltpu.VMEM((_CHUNK, EMBED_DIM), jnp.float32),
        pltpu.SemaphoreType.DMA,
    ],
)
def _gather_kernel(table_hbm, idx_hbm, out_hbm, table_v, idx_v, dummy_v, sem):
    cidx = lax.axis_index("c")
    wid = lax.axis_index("s") * _NC + cidx
    base = wid * _B_PER_W

    pltpu.sync_copy(idx_hbm.at[pl.ds(base, _B_PER_W)], idx_v)
    pltpu.sync_copy(table_hbm.at[wid], table_v)

    _NL = 16

    def body(g, _):
        lblv = idx_v[pl.ds(g * _NL, _NL)]
        for k in range(_NL):
            pltpu.async_copy(
                table_v.at[lblv[k]],
                out_hbm.at[base + g * _NL + k, 0],
                sem,
            )
        return 0

    lax.fori_loop(0, _B_PER_W // _NL, body, 0)
    for i in range(_B_PER_W // _CHUNK):
        pltpu.make_async_copy(
            out_hbm.at[pl.ds(base + i * _CHUNK, _CHUNK), 0], dummy_v, sem
        ).wait()


def kernel(params, labels, head_table):
    del params
    table_rep = jnp.broadcast_to(head_table[None], (_NW, NUM_HEAD, EMBED_DIM))
    return _gather_kernel(table_rep, labels)
